# baseline (speedup 1.0000x reference)
import jax
import jax.numpy as jnp
from jax.experimental import pallas as pl

N, E, F_IN, HID, VAL, SE, NC, K_MC = 10000, 320000, 128, 64, 64, 8, 40, 5
GAMMA, L1_LAMBDA, TAU = 1.0, 0.05, 0.5


def _log_kernel(p_ref, o_ref):
    o_ref[...] = jnp.log(p_ref[...] + 1e-12)


def _final_log(probs):
    return pl.pallas_call(
        _log_kernel,
        out_shape=jax.ShapeDtypeStruct(probs.shape, probs.dtype),
    )(probs)


def _gcn_exact(x, src, dst, W, b, n):
    loop = jnp.arange(n)
    s = jnp.concatenate([src, loop])
    d = jnp.concatenate([dst, loop])
    deg = jnp.zeros(n, jnp.float32).at[d].add(1.0)
    dinv = jnp.where(deg > 0, 1.0 / jnp.sqrt(deg), 0.0)
    norm = dinv[s] * dinv[d]
    h = x @ W.T
    out = jnp.zeros((n, W.shape[0]), jnp.float32).at[d].add(norm[:, None] * h[s])
    return out + b


def _gumbel_sign_exact(logits, key, tau=TAU):
    u = jax.random.uniform(key, logits.shape, minval=1e-10, maxval=1.0)
    g = -jnp.log(-jnp.log(u))
    y_soft = jax.nn.softmax((logits + g) / tau, axis=-1)
    y_hard = jax.nn.one_hot(jnp.argmax(y_soft, axis=-1), 3, dtype=jnp.float32)
    y = y_hard + y_soft - jax.lax.stop_gradient(y_soft)
    sign_values = jnp.array([-1.0, 0.0, 1.0], jnp.float32)
    return (y * sign_values).sum(axis=-1)


def kernel(x, edge_index, y, train_mask, params):
    del y, train_mask
    n = N
    row, col = edge_index[0], edge_index[1]

    deg = jnp.zeros(n, jnp.float32).at[col].add(1.0) + 1.0
    dinv = 1.0 / jnp.sqrt(deg)

    def gcn_fast(h_in, W, b):
        h = h_in @ W.T
        hp = dinv[:, None] * h
        agg = jnp.zeros((n, W.shape[0]), jnp.float32).at[col].add(hp[row])
        return dinv[:, None] * agg + (dinv * dinv)[:, None] * h + b

    h1 = jax.nn.relu(gcn_fast(x, params['bb_w1'], params['bb_b1']))
    h2 = gcn_fast(h1, params['bb_w2'], params['bb_b2']) + x @ params['proj_w'].T
    H0 = jax.nn.relu(h2)

    hs = jax.nn.relu(_gcn_exact(x, row, col, params['se_w1'], params['se_b1'], n))
    hs = _gcn_exact(hs, row, col, params['se_w2'], params['se_b2'], n)
    edge_feat = jnp.concatenate([hs[row], hs[col]], axis=-1)
    eh = jax.nn.relu(edge_feat @ params['em_w1'].T + params['em_b1'])
    edge_logits = eh @ params['em_w2'].T + params['em_b2']

    edge_probs = jax.nn.softmax(edge_logits, axis=-1)
    prior_log = jnp.log(jnp.full((3,), 1.0 / 3.0, jnp.float32) + 1e-12)
    kl_mean = (edge_probs * (jnp.log(edge_probs) - prior_log)).sum(axis=-1).mean()
    recon = jnp.log(edge_probs[:, 0] + edge_probs[:, 2] + 1e-12).mean()
    struct_loss = kl_mean - recon

    gk = jax.random.key(42)
    esigns = [_gumbel_sign_exact(edge_logits, jax.random.fold_in(gk, k))
              for k in range(K_MC)]

    def s2(H, esign, lp):
        aw1 = lp['aw1']
        P = H @ (aw1[:, :VAL] @ lp['wt']).T
        Q = H @ (aw1[:, VAL:2 * VAL] @ lp['wv']).T
        V = H @ lp['wv'].T
        scb = lp['semb'] @ aw1[:, 2 * VAL:].T + lp['ab1']
        sidx = jnp.clip(esign.astype(jnp.int32) + 1, 0, 2)
        h_e = jax.nn.relu(P[col] + Q[row] + scb[sidx])
        alpha = h_e @ lp['aw2'][0] + lp['ab2'][0]
        alpha = jnp.where(alpha > L1_LAMBDA, alpha - L1_LAMBDA,
                          jnp.where(alpha < -L1_LAMBDA, alpha + L1_LAMBDA, 0.0))
        sl = jnp.abs(alpha).mean()
        c = jnp.where(esign > 0, alpha,
                      jnp.where(esign < 0, -jnp.abs(alpha), 0.0))
        agg = jnp.zeros((n, VAL), jnp.float32).at[col].add(c[:, None] * V[row])
        H_new = agg @ lp['wout'].T + lp['bout'] + H @ lp['wself'].T + H
        return jax.nn.relu(H_new), sl

    probs_sum = jnp.zeros((n, NC), jnp.float32)
    sparse_losses = []
    for k in range(K_MC):
        H = H0
        sl = 0.0
        for lp in params['layers']:
            H, sl = s2(H, esigns[k], lp)
        logits = H @ params['cls_w'].T + params['cls_b']
        probs_sum = probs_sum + jax.nn.softmax(logits, axis=-1)
        sparse_losses.append(sl)

    probs_mc = probs_sum / K_MC
    logits_mc = _final_log(probs_mc)
    sparse_loss = jnp.stack(sparse_losses).mean()
    return (logits_mc, sparse_loss, struct_loss)


# speedup vs baseline: 1.1352x; 1.1352x over previous
import functools

import jax
import jax.numpy as jnp
from jax import lax
from jax.experimental import pallas as pl

N, E, F_IN, HID, VAL, SE, NC, K_MC = 10000, 320000, 128, 64, 64, 8, 40, 5
GAMMA, L1_LAMBDA, TAU = 1.0, 0.05, 0.5

NW = 16
PER_W = E // NW
B = 80
NB = PER_W // B
CJ = 25
NC_J = NB // CJ
ACC_N = 10240
RPS = ACC_N // 16
RCHUNK = 128


def _log_kernel(p_ref, o_ref):
    o_ref[...] = jnp.log(p_ref[...] + 1e-12)


def _final_log(probs):
    return pl.pallas_call(
        _log_kernel,
        out_shape=jax.ShapeDtypeStruct(probs.shape, probs.dtype),
    )(probs)


EBL = 3200
NBLK = E // EBL


def _edge_body(pg_ref, qg_ref, vg_ref, sgn_ref, scb_ref, aw2_ref,
               val_ref, aabs_ref):
    sgn = sgn_ref[...]
    scbT = scb_ref[...]
    aw2 = aw2_ref[...]
    m2 = sgn >= 1.0
    m0 = sgn <= -1.0
    scb_sel = jnp.where(m2, scbT[:, 2:3], jnp.where(m0, scbT[:, 0:1],
                                                    scbT[:, 1:2]))
    h = jnp.maximum(pg_ref[...] + qg_ref[...] + scb_sel, 0.0)
    alpha = jnp.sum(h * aw2[:, 0:1], axis=0, keepdims=True) + aw2[0, 1]
    alpha = jnp.where(alpha > L1_LAMBDA, alpha - L1_LAMBDA,
                      jnp.where(alpha < -L1_LAMBDA, alpha + L1_LAMBDA, 0.0))
    aabs_ref[...] = jnp.abs(alpha)
    c = jnp.where(sgn > 0.0, alpha, jnp.where(sgn < 0.0, -jnp.abs(alpha), 0.0))
    val_ref[...] = vg_ref[...] * c


def _edge_pipeline(PgT, QgT, VgT, sgn, scbT, aw2c):
    grid = (NBLK,)
    bs_feat = pl.BlockSpec((VAL, EBL), lambda g: (0, g))
    bs_sgn = pl.BlockSpec((1, EBL), lambda g: (0, g))
    bs_scb = pl.BlockSpec((VAL, 3), lambda g: (0, 0))
    bs_aw2 = pl.BlockSpec((VAL, 2), lambda g: (0, 0))
    valT, aabs = pl.pallas_call(
        _edge_body,
        grid=grid,
        in_specs=[bs_feat, bs_feat, bs_feat, bs_sgn, bs_scb, bs_aw2],
        out_specs=[bs_feat, bs_sgn],
        out_shape=[jax.ShapeDtypeStruct((VAL, E), jnp.float32),
                   jax.ShapeDtypeStruct((1, E), jnp.float32)],
    )(PgT, QgT, VgT, sgn, scbT, aw2c)
    return valT, aabs


def _gcn_exact(x, src, dst, W, b, n):
    loop = jnp.arange(n)
    s = jnp.concatenate([src, loop])
    d = jnp.concatenate([dst, loop])
    deg = jnp.zeros(n, jnp.float32).at[d].add(1.0)
    dinv = jnp.where(deg > 0, 1.0 / jnp.sqrt(deg), 0.0)
    norm = dinv[s] * dinv[d]
    h = x @ W.T
    out = jnp.zeros((n, W.shape[0]), jnp.float32).at[d].add(norm[:, None] * h[s])
    return out + b


def _gumbel_sign_exact(logits, key, tau=TAU):
    u = jax.random.uniform(key, logits.shape, minval=1e-10, maxval=1.0)
    g = -jnp.log(-jnp.log(u))
    y_soft = jax.nn.softmax((logits + g) / tau, axis=-1)
    y_hard = jax.nn.one_hot(jnp.argmax(y_soft, axis=-1), 3, dtype=jnp.float32)
    y = y_hard + y_soft - jax.lax.stop_gradient(y_soft)
    sign_values = jnp.array([-1.0, 0.0, 1.0], jnp.float32)
    return (y * sign_values).sum(axis=-1)


def kernel(x, edge_index, y, train_mask, params):
    del y, train_mask
    n = N
    row, col = edge_index[0], edge_index[1]

    deg = jnp.zeros(n, jnp.float32).at[col].add(1.0) + 1.0
    dinv = 1.0 / jnp.sqrt(deg)

    def gcn_fast(h_in, W, b):
        h = h_in @ W.T
        hp = dinv[:, None] * h
        agg = jnp.zeros((n, W.shape[0]), jnp.float32).at[col].add(hp[row])
        return dinv[:, None] * agg + (dinv * dinv)[:, None] * h + b

    h1 = jax.nn.relu(gcn_fast(x, params['bb_w1'], params['bb_b1']))
    h2 = gcn_fast(h1, params['bb_w2'], params['bb_b2']) + x @ params['proj_w'].T
    H0 = jax.nn.relu(h2)

    hs = jax.nn.relu(_gcn_exact(x, row, col, params['se_w1'], params['se_b1'], n))
    hs = _gcn_exact(hs, row, col, params['se_w2'], params['se_b2'], n)
    edge_feat = jnp.concatenate([hs[row], hs[col]], axis=-1)
    eh = jax.nn.relu(edge_feat @ params['em_w1'].T + params['em_b1'])
    edge_logits = eh @ params['em_w2'].T + params['em_b2']

    edge_probs = jax.nn.softmax(edge_logits, axis=-1)
    prior_log = jnp.log(jnp.full((3,), 1.0 / 3.0, jnp.float32) + 1e-12)
    kl_mean = (edge_probs * (jnp.log(edge_probs) - prior_log)).sum(axis=-1).mean()
    recon = jnp.log(edge_probs[:, 0] + edge_probs[:, 2] + 1e-12).mean()
    struct_loss = kl_mean - recon

    gk = jax.random.key(42)
    esigns = [_gumbel_sign_exact(edge_logits, jax.random.fold_in(gk, k))
              for k in range(K_MC)]

    wps = []
    for lp in params['layers']:
        aw1 = lp['aw1']
        scb = lp['semb'] @ aw1[:, 2 * VAL:].T + lp['ab1']
        wp = jnp.zeros((8, VAL), jnp.float32)
        wp = wp.at[0:3].set(scb)
        wp = wp.at[3].set(lp['aw2'][0])
        wp = wp.at[4].set(jnp.full((VAL,), lp['ab2'][0] / 16.0))
        wps.append(wp)

    def s2(H, sgn4, lp, wp):
        aw1 = lp['aw1']
        P = H @ (aw1[:, :VAL] @ lp['wt']).T
        Q = H @ (aw1[:, VAL:2 * VAL] @ lp['wv']).T
        V = H @ lp['wv'].T
        esign = sgn4
        scb = lp['semb'] @ aw1[:, 2 * VAL:].T + lp['ab1']
        PgT = P.T[:, col]
        QgT = Q.T[:, row]
        VgT = V.T[:, row]
        aw2c = jnp.stack([lp['aw2'][0],
                          jnp.full((VAL,), lp['ab2'][0])], axis=1)
        valT, aabs = _edge_pipeline(PgT, QgT, VgT, esign[None, :], scb.T, aw2c)
        sl = aabs.sum() / E
        agg = jnp.zeros((N, VAL), jnp.float32).at[col].add(valT.T)
        H_new = agg @ lp['wout'].T + lp['bout'] + H @ lp['wself'].T + H
        return jax.nn.relu(H_new), sl

    probs_sum = jnp.zeros((n, NC), jnp.float32)
    sparse_losses = []
    for k in range(K_MC):
        sgn4 = esigns[k]
        H = H0
        sl = 0.0
        for li, lp in enumerate(params['layers']):
            H, sl = s2(H, sgn4, lp, wps[li])
        logits = H @ params['cls_w'].T + params['cls_b']
        probs_sum = probs_sum + jax.nn.softmax(logits, axis=-1)
        sparse_losses.append(sl)

    probs_mc = probs_sum / K_MC
    logits_mc = _final_log(probs_mc)
    sparse_loss = jnp.stack(sparse_losses).mean()
    return (logits_mc, sparse_loss, struct_loss)


# speedup vs baseline: 1.1353x; 1.0000x over previous
import jax
import jax.numpy as jnp
from jax import lax
from jax.experimental import pallas as pl

N, E, F_IN, HID, VAL, SE, NC, K_MC = 10000, 320000, 128, 64, 64, 8, 40, 5
GAMMA, L1_LAMBDA, TAU = 1.0, 0.05, 0.5

NW = 16
PER_W = E // NW
B = 80
NB = PER_W // B
CJ = 25
NC_J = NB // CJ
ACC_N = 10240
RPS = ACC_N // 16
RCHUNK = 128


def _log_kernel(p_ref, o_ref):
    o_ref[...] = jnp.log(p_ref[...] + 1e-12)


def _final_log(probs):
    return pl.pallas_call(
        _log_kernel,
        out_shape=jax.ShapeDtypeStruct(probs.shape, probs.dtype),
    )(probs)


EBL = 3200
NBLK = E // EBL


def _edge_body(pg_ref, qg_ref, vg_ref, sgn_ref, scb_ref, aw2_ref,
               val_ref, aabs_ref):
    sgn = sgn_ref[...]
    scbT = scb_ref[...]
    aw2 = aw2_ref[...]
    m2 = sgn >= 1.0
    m0 = sgn <= -1.0
    scb_sel = jnp.where(m2, scbT[:, 2:3], jnp.where(m0, scbT[:, 0:1],
                                                    scbT[:, 1:2]))
    h = jnp.maximum(pg_ref[...] + qg_ref[...] + scb_sel, 0.0)
    alpha = jnp.sum(h * aw2[:, 0:1], axis=0, keepdims=True) + aw2[0, 1]
    alpha = jnp.where(alpha > L1_LAMBDA, alpha - L1_LAMBDA,
                      jnp.where(alpha < -L1_LAMBDA, alpha + L1_LAMBDA, 0.0))
    aabs_ref[...] = jnp.abs(alpha)
    c = jnp.where(sgn > 0.0, alpha, jnp.where(sgn < 0.0, -jnp.abs(alpha), 0.0))
    val_ref[...] = vg_ref[...] * c


def _edge_pipeline(PgT, QgT, VgT, sgn, scbT, aw2c):
    grid = (NBLK,)
    bs_feat = pl.BlockSpec((VAL, EBL), lambda g: (0, g))
    bs_sgn = pl.BlockSpec((1, EBL), lambda g: (0, g))
    bs_scb = pl.BlockSpec((VAL, 3), lambda g: (0, 0))
    bs_aw2 = pl.BlockSpec((VAL, 2), lambda g: (0, 0))
    valT, aabs = pl.pallas_call(
        _edge_body,
        grid=grid,
        in_specs=[bs_feat, bs_feat, bs_feat, bs_sgn, bs_scb, bs_aw2],
        out_specs=[bs_feat, bs_sgn],
        out_shape=[jax.ShapeDtypeStruct((VAL, E), jnp.float32),
                   jax.ShapeDtypeStruct((1, E), jnp.float32)],
    )(PgT, QgT, VgT, sgn, scbT, aw2c)
    return valT, aabs


def _gcn_exact(x, src, dst, W, b, n):
    loop = jnp.arange(n)
    s = jnp.concatenate([src, loop])
    d = jnp.concatenate([dst, loop])
    deg = jnp.zeros(n, jnp.float32).at[d].add(1.0)
    dinv = jnp.where(deg > 0, 1.0 / jnp.sqrt(deg), 0.0)
    norm = dinv[s] * dinv[d]
    h = x @ W.T
    out = jnp.zeros((n, W.shape[0]), jnp.float32).at[d].add(norm[:, None] * h[s])
    return out + b


def _gumbel_sign_exact(logits, key, tau=TAU):
    u = jax.random.uniform(key, logits.shape, minval=1e-10, maxval=1.0)
    g = -jnp.log(-jnp.log(u))
    y_soft = jax.nn.softmax((logits + g) / tau, axis=-1)
    y_hard = jax.nn.one_hot(jnp.argmax(y_soft, axis=-1), 3, dtype=jnp.float32)
    y = y_hard + y_soft - jax.lax.stop_gradient(y_soft)
    sign_values = jnp.array([-1.0, 0.0, 1.0], jnp.float32)
    return (y * sign_values).sum(axis=-1)


def kernel(x, edge_index, y, train_mask, params):
    del y, train_mask
    n = N
    row, col = edge_index[0], edge_index[1]

    deg = jnp.zeros(n, jnp.float32).at[col].add(1.0) + 1.0
    dinv = 1.0 / jnp.sqrt(deg)

    def gcn_fast(h_in, W, b):
        h = h_in @ W.T
        hp = dinv[:, None] * h
        agg = jnp.zeros((n, W.shape[0]), jnp.float32).at[col].add(hp[row])
        return dinv[:, None] * agg + (dinv * dinv)[:, None] * h + b

    h1 = jax.nn.relu(gcn_fast(x, params['bb_w1'], params['bb_b1']))
    h2 = gcn_fast(h1, params['bb_w2'], params['bb_b2']) + x @ params['proj_w'].T
    H0 = jax.nn.relu(h2)

    hs = jax.nn.relu(_gcn_exact(x, row, col, params['se_w1'], params['se_b1'], n))
    hs = _gcn_exact(hs, row, col, params['se_w2'], params['se_b2'], n)
    edge_feat = jnp.concatenate([hs[row], hs[col]], axis=-1)
    eh = jax.nn.relu(edge_feat @ params['em_w1'].T + params['em_b1'])
    edge_logits = eh @ params['em_w2'].T + params['em_b2']

    edge_probs = jax.nn.softmax(edge_logits, axis=-1)
    prior_log = jnp.log(jnp.full((3,), 1.0 / 3.0, jnp.float32) + 1e-12)
    kl_mean = (edge_probs * (jnp.log(edge_probs) - prior_log)).sum(axis=-1).mean()
    recon = jnp.log(edge_probs[:, 0] + edge_probs[:, 2] + 1e-12).mean()
    struct_loss = kl_mean - recon

    gk = jax.random.key(42)
    esigns = [_gumbel_sign_exact(edge_logits, jax.random.fold_in(gk, k))
              for k in range(K_MC)]

    wps = []
    for lp in params['layers']:
        aw1 = lp['aw1']
        scb = lp['semb'] @ aw1[:, 2 * VAL:].T + lp['ab1']
        wp = jnp.zeros((8, VAL), jnp.float32)
        wp = wp.at[0:3].set(scb)
        wp = wp.at[3].set(lp['aw2'][0])
        wp = wp.at[4].set(jnp.full((VAL,), lp['ab2'][0] / 16.0))
        wps.append(wp)

    def s2(H, sgn4, lp, wp):
        aw1 = lp['aw1']
        P = H @ (aw1[:, :VAL] @ lp['wt']).T
        Q = H @ (aw1[:, VAL:2 * VAL] @ lp['wv']).T
        V = H @ lp['wv'].T
        esign = sgn4
        scb = lp['semb'] @ aw1[:, 2 * VAL:].T + lp['ab1']
        PgT = P.T[:, col]
        QgT = Q.T[:, row]
        VgT = V.T[:, row]
        aw2c = jnp.stack([lp['aw2'][0],
                          jnp.full((VAL,), lp['ab2'][0])], axis=1)
        valT, aabs = _edge_pipeline(PgT, QgT, VgT, esign[None, :], scb.T, aw2c)
        sl = aabs.sum() / E
        agg = jnp.zeros((N, VAL), jnp.float32).at[col].add(valT.T)
        H_new = agg @ lp['wout'].T + lp['bout'] + H @ lp['wself'].T + H
        return jax.nn.relu(H_new), sl

    probs_sum = jnp.zeros((n, NC), jnp.float32)
    sparse_losses = []
    for k in range(K_MC):
        sgn4 = esigns[k]
        H = H0
        sl = 0.0
        for li, lp in enumerate(params['layers']):
            H, sl = s2(H, sgn4, lp, wps[li])
        logits = H @ params['cls_w'].T + params['cls_b']
        probs_sum = probs_sum + jax.nn.softmax(logits, axis=-1)
        sparse_losses.append(sl)

    probs_mc = probs_sum / K_MC
    logits_mc = _final_log(probs_mc)
    sparse_loss = jnp.stack(sparse_losses).mean()
    return (logits_mc, sparse_loss, struct_loss)
